# baseline (speedup 1.0000x reference)
import functools

import jax
import jax.numpy as jnp
from jax import lax
from jax.experimental import pallas as pl
from jax.experimental.pallas import tpu as pltpu
from jax.experimental.pallas import tpu_sc as plsc

NUM_EXPERTS = 8
TOP_K = 2
HIDDEN = 768
INTER = 2048
T = 2048
NP = T * TOP_K
BLK = 512
MAXB = 16
PMAX = MAXB * BLK

_SC_INFO = plsc.get_sparse_core_info()
_NC, _NS, _L = _SC_INFO.num_cores, _SC_INFO.num_subcores, _SC_INFO.num_lanes
_NW = _NC * _NS


def _router_body(x_ref, wr_ref, pos_ref, vflat_ref, meta_ref):
    x = x_ref[...]
    logits = lax.dot_general(x, wr_ref[...], (((1,), (0,)), ((), ())),
                             preferred_element_type=jnp.float32)
    aff = jax.nn.sigmoid(logits)
    eiota = lax.broadcasted_iota(jnp.int32, (T, NUM_EXPERTS), 1)
    m1 = jnp.max(aff, axis=-1, keepdims=True)
    im1 = jnp.min(jnp.where(aff == m1, eiota, NUM_EXPERTS), axis=-1, keepdims=True)
    aff2 = jnp.where(eiota == im1, -jnp.inf, aff)
    m2 = jnp.max(aff2, axis=-1, keepdims=True)
    im2 = jnp.min(jnp.where(aff2 == m2, eiota, NUM_EXPERTS), axis=-1, keepdims=True)
    s = m1 + m2
    v0 = m1 / s
    v1 = m2 / s
    vflat_ref[...] = jnp.concatenate([v0, v1], axis=0)

    idx_full = jnp.concatenate([im1, im2], axis=0)
    piota = lax.broadcasted_iota(jnp.int32, (NP, NUM_EXPERTS), 1)
    onehot = (piota == idx_full).astype(jnp.float32)

    counts = jnp.sum(onehot, axis=0, keepdims=True).astype(jnp.int32)
    blocks = (counts + (BLK - 1)) >> 9
    li = lax.broadcasted_iota(jnp.int32, (NUM_EXPERTS, NUM_EXPERTS), 0)
    lj = lax.broadcasted_iota(jnp.int32, (NUM_EXPERTS, NUM_EXPERTS), 1)
    l8_incl = (li <= lj).astype(jnp.float32)
    cum_incl = lax.dot_general(blocks.astype(jnp.float32), l8_incl,
                               (((1,), (0,)), ((), ())),
                               preferred_element_type=jnp.float32).astype(jnp.int32)
    pad_off = (cum_incl - blocks) * BLK

    used = jnp.max(cum_incl)
    e_sub = lax.broadcasted_iota(jnp.int32, (1, NUM_EXPERTS), 1)
    lastexp = jnp.max(jnp.where(blocks > 0, e_sub, -1))
    counts_sub = lax.dot_general(
        onehot, jnp.ones((NP, 1), dtype=jnp.float32), (((0,), (0,)), ((), ())),
        preferred_element_type=jnp.float32).astype(jnp.int32)
    blocks_sub = (counts_sub + (BLK - 1)) >> 9
    li2 = lax.broadcasted_iota(jnp.int32, (NUM_EXPERTS, NUM_EXPERTS), 0)
    lj2 = lax.broadcasted_iota(jnp.int32, (NUM_EXPERTS, NUM_EXPERTS), 1)
    l8_incl_sub = (lj2 <= li2).astype(jnp.float32)
    cum_incl_sub = lax.dot_general(
        l8_incl_sub, blocks_sub.astype(jnp.float32), (((1,), (0,)), ((), ())),
        preferred_element_type=jnp.float32).astype(jnp.int32)
    biota = lax.broadcasted_iota(jnp.int32, (NUM_EXPERTS, 128), 1)
    be_raw = jnp.sum((cum_incl_sub <= biota).astype(jnp.int32), axis=0,
                     keepdims=True)
    meta_ref[0:1, :] = jnp.minimum(be_raw, lastexp)
    meta_ref[1:2, :] = jnp.full((1, 128), 0, jnp.int32) + used

    pad_sel = jnp.sum(onehot * pad_off.astype(jnp.float32), axis=1,
                      keepdims=True)
    tile = 512
    for tnum in range(NP // tile):
        gi = lax.broadcasted_iota(jnp.int32, (tile, NP), 0) + tnum * tile
        gj = lax.broadcasted_iota(jnp.int32, (tile, NP), 1)
        ltri = (gj < gi).astype(jnp.float32)
        rank_t = lax.dot_general(ltri, onehot, (((1,), (0,)), ((), ())),
                                 preferred_element_type=jnp.float32)
        oh_t = onehot[tnum * tile:(tnum + 1) * tile, :]
        rank_sel = jnp.sum(oh_t * rank_t, axis=1, keepdims=True)
        pos_t = pad_sel[tnum * tile:(tnum + 1) * tile, :] + rank_sel
        pos_ref[tnum * tile:(tnum + 1) * tile, :] = pos_t.astype(jnp.int32)


def _run_router(xf, w_router):
    return pl.pallas_call(
        _router_body,
        out_shape=(
            jax.ShapeDtypeStruct((NP, 1), jnp.int32),
            jax.ShapeDtypeStruct((NP, 1), jnp.float32),
            jax.ShapeDtypeStruct((2, 128), jnp.int32),
        ),
    )(xf, w_router)


_PPW = NP // _NW


def _scatter_body(pos_hbm, vflat_hbm, tid_hbm, val_hbm, posv, vv, tidv, sem):
    wid = lax.axis_index("s") * _NC + lax.axis_index("c")
    base = wid * _PPW
    pltpu.sync_copy(pos_hbm.at[pl.ds(base, _PPW)], posv)
    pltpu.sync_copy(vflat_hbm.at[pl.ds(base, _PPW)], vv)
    lane = lax.broadcasted_iota(jnp.int32, (_L,), 0)
    for i in range(_PPW // _L):
        tidv[pl.ds(i * _L, _L)] = (lane + (base + i * _L)) & (T - 1)
    pltpu.async_copy(tidv, tid_hbm.at[posv], sem).wait()
    pltpu.async_copy(vv, val_hbm.at[posv], sem).wait()


@functools.partial(
    pl.kernel,
    mesh=plsc.VectorSubcoreMesh(core_axis_name="c", subcore_axis_name="s"),
    out_type=(
        jax.ShapeDtypeStruct((PMAX,), jnp.int32),
        jax.ShapeDtypeStruct((PMAX,), jnp.float32),
    ),
    scratch_types=[
        pltpu.VMEM((_PPW,), jnp.int32),
        pltpu.VMEM((_PPW,), jnp.float32),
        pltpu.VMEM((_PPW,), jnp.int32),
        pltpu.SemaphoreType.DMA,
    ],
)
def _sc_scatter(pos_hbm, vflat_hbm, tid_hbm, val_hbm, posv, vv, tidv, sem):
    _scatter_body(pos_hbm, vflat_hbm, tid_hbm, val_hbm, posv, vv, tidv, sem)


_SPW = PMAX // _NW
_GCH = 64


def _gather_body(tid_hbm, xf_hbm, xs_hbm, tidv, rows, sem):
    wid = lax.axis_index("s") * _NC + lax.axis_index("c")
    for c in range(_SPW // _GCH):
        base = wid * _SPW + c * _GCH
        pltpu.sync_copy(tid_hbm.at[pl.ds(base, _GCH)], tidv)
        for i in range(_GCH // _L):
            sl = pl.ds(i * _L, _L)
            tidv[sl] = jnp.minimum(jnp.maximum(tidv[sl], 0), T - 1)
        pltpu.async_copy(xf_hbm.at[tidv], rows, sem).wait()
        pltpu.sync_copy(rows, xs_hbm.at[pl.ds(base, _GCH)])


@functools.partial(
    pl.kernel,
    mesh=plsc.VectorSubcoreMesh(core_axis_name="c", subcore_axis_name="s"),
    out_type=jax.ShapeDtypeStruct((PMAX, HIDDEN), jnp.float32),
    scratch_types=[
        pltpu.VMEM((_GCH,), jnp.int32),
        pltpu.VMEM((_GCH, HIDDEN), jnp.float32),
        pltpu.SemaphoreType.DMA,
    ],
)
def _sc_gather(tid_hbm, xf_hbm, xs_hbm, tidv, rows, sem):
    _gather_body(tid_hbm, xf_hbm, xs_hbm, tidv, rows, sem)


def _ffn_body(bexp_ref, used_ref, x_ref, wg_ref, wu_ref, wd_ref, val_ref,
              out_ref):
    b = pl.program_id(0)

    @pl.when(b < used_ref[0])
    def _compute():
        x = x_ref[...]
        g = lax.dot_general(x, wg_ref[0], (((1,), (0,)), ((), ())),
                            preferred_element_type=jnp.float32)
        u = lax.dot_general(x, wu_ref[0], (((1,), (0,)), ((), ())),
                            preferred_element_type=jnp.float32)
        h = (g * jax.nn.sigmoid(g)) * u
        h = h * val_ref[...]
        out_ref[...] = lax.dot_general(h, wd_ref[0], (((1,), (0,)), ((), ())),
                                       preferred_element_type=jnp.float32)


def _run_ffn(bexp, used, x_sorted, w_gate, w_up, w_down, val_col):
    def clamp(b, u):
        return jnp.minimum(b, u[0] - 1)

    grid_spec = pltpu.PrefetchScalarGridSpec(
        num_scalar_prefetch=2,
        grid=(MAXB,),
        in_specs=[
            pl.BlockSpec((BLK, HIDDEN), lambda b, be, u: (clamp(b, u), 0)),
            pl.BlockSpec((1, HIDDEN, INTER),
                         lambda b, be, u: (be[clamp(b, u)], 0, 0)),
            pl.BlockSpec((1, HIDDEN, INTER),
                         lambda b, be, u: (be[clamp(b, u)], 0, 0)),
            pl.BlockSpec((1, INTER, HIDDEN),
                         lambda b, be, u: (be[clamp(b, u)], 0, 0)),
            pl.BlockSpec((BLK, 1), lambda b, be, u: (clamp(b, u), 0)),
        ],
        out_specs=pl.BlockSpec((BLK, HIDDEN), lambda b, be, u: (clamp(b, u), 0)),
    )
    return pl.pallas_call(
        _ffn_body,
        grid_spec=grid_spec,
        out_shape=jax.ShapeDtypeStruct((PMAX, HIDDEN), jnp.float32),
    )(bexp, used, x_sorted, w_gate, w_up, w_down, val_col)


_TPW = T // _NW


def _combine_body(pos_hbm, ys_hbm, out_hbm, p0v, p1v, r0, r1, sem):
    wid = lax.axis_index("s") * _NC + lax.axis_index("c")
    tbase = wid * _TPW
    pltpu.sync_copy(pos_hbm.at[pl.ds(tbase, _TPW)], p0v)
    pltpu.sync_copy(pos_hbm.at[pl.ds(T + tbase, _TPW)], p1v)
    pltpu.async_copy(ys_hbm.at[p0v], r0, sem).wait()
    pltpu.async_copy(ys_hbm.at[p1v], r1, sem).wait()

    def row_add(r, _):
        def col_add(c, _):
            sl = pl.ds(c * _L, _L)
            r0[r, sl] = r0[r, sl] + r1[r, sl]
            return 0
        return lax.fori_loop(0, HIDDEN // _L, col_add, 0)

    lax.fori_loop(0, _TPW, row_add, 0)
    pltpu.sync_copy(r0, out_hbm.at[pl.ds(tbase, _TPW)])


@functools.partial(
    pl.kernel,
    mesh=plsc.VectorSubcoreMesh(core_axis_name="c", subcore_axis_name="s"),
    out_type=jax.ShapeDtypeStruct((T, HIDDEN), jnp.float32),
    scratch_types=[
        pltpu.VMEM((_TPW,), jnp.int32),
        pltpu.VMEM((_TPW,), jnp.int32),
        pltpu.VMEM((_TPW, HIDDEN), jnp.float32),
        pltpu.VMEM((_TPW, HIDDEN), jnp.float32),
        pltpu.SemaphoreType.DMA,
    ],
)
def _sc_combine(pos_hbm, ys_hbm, out_hbm, p0v, p1v, r0, r1, sem):
    _combine_body(pos_hbm, ys_hbm, out_hbm, p0v, p1v, r0, r1, sem)


@jax.jit
def kernel(hidden_states, w_router, w_gate, w_up, w_down):
    b, s, hd = hidden_states.shape
    xf = hidden_states.reshape(b * s, hd)

    pos_col, vflat_col, meta = _run_router(xf, w_router)
    pos = pos_col.reshape(NP)
    vflat = vflat_col.reshape(NP)
    bexp = meta[0, :MAXB]
    used = meta[1, :1]

    tid_sorted, val_sorted = _sc_scatter(pos, vflat)
    x_sorted = _sc_gather(tid_sorted, xf)
    y_sorted = _run_ffn(bexp, used, x_sorted, w_gate, w_up, w_down,
                        val_sorted.reshape(PMAX, 1))
    out = _sc_combine(pos, y_sorted)
    return out.reshape(b, s, hd)


# speedup vs baseline: 1.4141x; 1.4141x over previous
import functools

import jax
import jax.numpy as jnp
from jax import lax
from jax.experimental import pallas as pl
from jax.experimental.pallas import tpu as pltpu
from jax.experimental.pallas import tpu_sc as plsc

NUM_EXPERTS = 8
TOP_K = 2
HIDDEN = 768
INTER = 2048
T = 2048
NP = T * TOP_K
BLK = 512
MAXB = 16
PMAX = MAXB * BLK

_SC_INFO = plsc.get_sparse_core_info()
_NC, _NS, _L = _SC_INFO.num_cores, _SC_INFO.num_subcores, _SC_INFO.num_lanes
_NW = _NC * _NS


def _router_body(x_ref, wr_ref, pos_ref, vflat_ref, meta_ref):
    x = x_ref[...]
    logits = lax.dot_general(x, wr_ref[...], (((1,), (0,)), ((), ())),
                             preferred_element_type=jnp.float32)
    aff = jax.nn.sigmoid(logits)
    eiota = lax.broadcasted_iota(jnp.int32, (T, NUM_EXPERTS), 1)
    m1 = jnp.max(aff, axis=-1, keepdims=True)
    im1 = jnp.min(jnp.where(aff == m1, eiota, NUM_EXPERTS), axis=-1, keepdims=True)
    aff2 = jnp.where(eiota == im1, -jnp.inf, aff)
    m2 = jnp.max(aff2, axis=-1, keepdims=True)
    im2 = jnp.min(jnp.where(aff2 == m2, eiota, NUM_EXPERTS), axis=-1, keepdims=True)
    s = m1 + m2
    v0 = m1 / s
    v1 = m2 / s
    vflat_ref[...] = jnp.concatenate([v0, v1], axis=0)

    idx_full = jnp.concatenate([im1, im2], axis=0)
    piota = lax.broadcasted_iota(jnp.int32, (NP, NUM_EXPERTS), 1)
    onehot = (piota == idx_full).astype(jnp.float32)

    counts = jnp.sum(onehot, axis=0, keepdims=True).astype(jnp.int32)
    blocks = (counts + (BLK - 1)) >> 9
    li = lax.broadcasted_iota(jnp.int32, (NUM_EXPERTS, NUM_EXPERTS), 0)
    lj = lax.broadcasted_iota(jnp.int32, (NUM_EXPERTS, NUM_EXPERTS), 1)
    l8_incl = (li <= lj).astype(jnp.float32)
    cum_incl = lax.dot_general(blocks.astype(jnp.float32), l8_incl,
                               (((1,), (0,)), ((), ())),
                               preferred_element_type=jnp.float32).astype(jnp.int32)
    pad_off = (cum_incl - blocks) * BLK

    used = jnp.max(cum_incl)
    e_sub = lax.broadcasted_iota(jnp.int32, (1, NUM_EXPERTS), 1)
    lastexp = jnp.max(jnp.where(blocks > 0, e_sub, -1))
    counts_sub = lax.dot_general(
        onehot, jnp.ones((NP, 1), dtype=jnp.float32), (((0,), (0,)), ((), ())),
        preferred_element_type=jnp.float32).astype(jnp.int32)
    blocks_sub = (counts_sub + (BLK - 1)) >> 9
    li2 = lax.broadcasted_iota(jnp.int32, (NUM_EXPERTS, NUM_EXPERTS), 0)
    lj2 = lax.broadcasted_iota(jnp.int32, (NUM_EXPERTS, NUM_EXPERTS), 1)
    l8_incl_sub = (lj2 <= li2).astype(jnp.float32)
    cum_incl_sub = lax.dot_general(
        l8_incl_sub, blocks_sub.astype(jnp.float32), (((1,), (0,)), ((), ())),
        preferred_element_type=jnp.float32).astype(jnp.int32)
    biota = lax.broadcasted_iota(jnp.int32, (NUM_EXPERTS, 128), 1)
    be_raw = jnp.sum((cum_incl_sub <= biota).astype(jnp.int32), axis=0,
                     keepdims=True)
    meta_ref[0:1, :] = jnp.minimum(be_raw, lastexp)
    meta_ref[1:2, :] = jnp.full((1, 128), 0, jnp.int32) + used

    pad_sel = jnp.sum(onehot * pad_off.astype(jnp.float32), axis=1,
                      keepdims=True)
    tile = 512
    for tnum in range(NP // tile):
        gi = lax.broadcasted_iota(jnp.int32, (tile, NP), 0) + tnum * tile
        gj = lax.broadcasted_iota(jnp.int32, (tile, NP), 1)
        ltri = (gj < gi).astype(jnp.float32)
        rank_t = lax.dot_general(ltri, onehot, (((1,), (0,)), ((), ())),
                                 preferred_element_type=jnp.float32)
        oh_t = onehot[tnum * tile:(tnum + 1) * tile, :]
        rank_sel = jnp.sum(oh_t * rank_t, axis=1, keepdims=True)
        pos_t = pad_sel[tnum * tile:(tnum + 1) * tile, :] + rank_sel
        pos_ref[tnum * tile:(tnum + 1) * tile, :] = pos_t.astype(jnp.int32)


def _run_router(xf, w_router):
    return pl.pallas_call(
        _router_body,
        out_shape=(
            jax.ShapeDtypeStruct((NP, 1), jnp.int32),
            jax.ShapeDtypeStruct((NP, 1), jnp.float32),
            jax.ShapeDtypeStruct((2, 128), jnp.int32),
        ),
    )(xf, w_router)


_PPS = NP // _NS
_SPW = PMAX // _NW
_GCH = 64


def _dispatch_body(pos_hbm, vflat_hbm, xf_hbm, xs_hbm, val_hbm,
                   spm_tid, spm_val, posv, vv, tidv,
                   idx0, idx1, rows0, rows1,
                   gsem0, gsem1, wsem0, wsem1):
    cid = lax.axis_index("c")
    sid = lax.axis_index("s")
    wid = sid * _NC + cid

    lane = lax.broadcasted_iota(jnp.int32, (_L,), 0)
    for r in range(_PPS // 128):
        base = sid * _PPS + r * 128
        pltpu.sync_copy(pos_hbm.at[pl.ds(base, 128)], posv)
        pltpu.sync_copy(vflat_hbm.at[pl.ds(base, 128)], vv)
        for i in range(128 // _L):
            tidv[pl.ds(i * _L, _L)] = (lane + (base + i * _L)) & (T - 1)
        pltpu.sync_copy(tidv, spm_tid.at[posv])
        pltpu.sync_copy(vv, spm_val.at[posv])
    plsc.subcore_barrier()

    srange = wid * _SPW
    pltpu.sync_copy(spm_val.at[pl.ds(srange, _SPW)],
                    val_hbm.at[pl.ds(srange, _SPW)])

    idxb = (idx0, idx1)
    rows = (rows0, rows1)
    gsem = (gsem0, gsem1)
    wsem = (wsem0, wsem1)
    nch = _SPW // _GCH
    g = [None, None]
    w = [None, None]
    for c in range(nch):
        b = c & 1
        if w[b] is not None:
            w[b].wait()
        cbase = srange + c * _GCH
        pltpu.sync_copy(spm_tid.at[pl.ds(cbase, _GCH)], idxb[b])
        for i in range(_GCH // _L):
            sl = pl.ds(i * _L, _L)
            idxb[b][sl] = jnp.minimum(jnp.maximum(idxb[b][sl], 0), T - 1)
        g[b] = pltpu.async_copy(xf_hbm.at[idxb[b]], rows[b], gsem[b])
        if c >= 1:
            pb = 1 - b
            g[pb].wait()
            w[pb] = pltpu.async_copy(
                rows[pb], xs_hbm.at[pl.ds(srange + (c - 1) * _GCH, _GCH)],
                wsem[pb])
    lb = (nch - 1) & 1
    g[lb].wait()
    w[lb] = pltpu.async_copy(
        rows[lb], xs_hbm.at[pl.ds(srange + (nch - 1) * _GCH, _GCH)], wsem[lb])
    w[0].wait()
    w[1].wait()


@functools.partial(
    pl.kernel,
    mesh=plsc.VectorSubcoreMesh(core_axis_name="c", subcore_axis_name="s"),
    out_type=(
        jax.ShapeDtypeStruct((PMAX, HIDDEN), jnp.float32),
        jax.ShapeDtypeStruct((PMAX,), jnp.float32),
    ),
    scratch_types=[
        pltpu.VMEM_SHARED((PMAX,), jnp.int32),
        pltpu.VMEM_SHARED((PMAX,), jnp.float32),
        pltpu.VMEM((128,), jnp.int32),
        pltpu.VMEM((128,), jnp.float32),
        pltpu.VMEM((128,), jnp.int32),
        pltpu.VMEM((_GCH,), jnp.int32),
        pltpu.VMEM((_GCH,), jnp.int32),
        pltpu.VMEM((_GCH, HIDDEN), jnp.float32),
        pltpu.VMEM((_GCH, HIDDEN), jnp.float32),
        pltpu.SemaphoreType.DMA,
        pltpu.SemaphoreType.DMA,
        pltpu.SemaphoreType.DMA,
        pltpu.SemaphoreType.DMA,
    ],
)
def _sc_dispatch(pos_hbm, vflat_hbm, xf_hbm, xs_hbm, val_hbm,
                 spm_tid, spm_val, posv, vv, tidv, idx0, idx1, rows0, rows1,
                 gsem0, gsem1, wsem0, wsem1):
    _dispatch_body(pos_hbm, vflat_hbm, xf_hbm, xs_hbm, val_hbm,
                   spm_tid, spm_val, posv, vv, tidv, idx0, idx1, rows0, rows1,
                   gsem0, gsem1, wsem0, wsem1)


def _ffn_body(bexp_ref, used_ref, x_ref, wg_ref, wu_ref, wd_ref, val_ref,
              out_ref):
    b = pl.program_id(0)

    @pl.when(b < used_ref[0])
    def _compute():
        x = x_ref[...]
        g = lax.dot_general(x, wg_ref[0], (((1,), (0,)), ((), ())),
                            preferred_element_type=jnp.float32)
        u = lax.dot_general(x, wu_ref[0], (((1,), (0,)), ((), ())),
                            preferred_element_type=jnp.float32)
        h = (g * jax.nn.sigmoid(g)) * u
        h = h * val_ref[...]
        out_ref[...] = lax.dot_general(h, wd_ref[0], (((1,), (0,)), ((), ())),
                                       preferred_element_type=jnp.float32)


def _run_ffn(bexp, used, x_sorted, w_gate, w_up, w_down, val_col):
    def clamp(b, u):
        return jnp.minimum(b, u[0] - 1)

    grid_spec = pltpu.PrefetchScalarGridSpec(
        num_scalar_prefetch=2,
        grid=(MAXB,),
        in_specs=[
            pl.BlockSpec((BLK, HIDDEN), lambda b, be, u: (clamp(b, u), 0)),
            pl.BlockSpec((1, HIDDEN, INTER),
                         lambda b, be, u: (be[clamp(b, u)], 0, 0)),
            pl.BlockSpec((1, HIDDEN, INTER),
                         lambda b, be, u: (be[clamp(b, u)], 0, 0)),
            pl.BlockSpec((1, INTER, HIDDEN),
                         lambda b, be, u: (be[clamp(b, u)], 0, 0)),
            pl.BlockSpec((BLK, 1), lambda b, be, u: (clamp(b, u), 0)),
        ],
        out_specs=pl.BlockSpec((BLK, HIDDEN), lambda b, be, u: (clamp(b, u), 0)),
    )
    return pl.pallas_call(
        _ffn_body,
        grid_spec=grid_spec,
        out_shape=jax.ShapeDtypeStruct((PMAX, HIDDEN), jnp.float32),
    )(bexp, used, x_sorted, w_gate, w_up, w_down, val_col)


_TPW = T // _NW
_CCH = 32


def _combine_body(pos_hbm, ys_hbm, out_hbm, p0a, p1a, p0b, p1b,
                  rA0, rA1, rB0, rB1,
                  sa0, sa1, sb0, sb1, swa, swb):
    wid = lax.axis_index("s") * _NC + lax.axis_index("c")
    tbase = wid * _TPW
    pltpu.sync_copy(pos_hbm.at[pl.ds(tbase, _CCH)], p0a)
    pltpu.sync_copy(pos_hbm.at[pl.ds(T + tbase, _CCH)], p1a)
    ga0 = pltpu.async_copy(ys_hbm.at[p0a], rA0, sa0)
    ga1 = pltpu.async_copy(ys_hbm.at[p1a], rA1, sa1)
    pltpu.sync_copy(pos_hbm.at[pl.ds(tbase + _CCH, _CCH)], p0b)
    pltpu.sync_copy(pos_hbm.at[pl.ds(T + tbase + _CCH, _CCH)], p1b)
    gb0 = pltpu.async_copy(ys_hbm.at[p0b], rB0, sb0)
    gb1 = pltpu.async_copy(ys_hbm.at[p1b], rB1, sb1)

    def add_rows(dst, src):
        def row_add(r, _):
            for c in range(HIDDEN // _L):
                sl = pl.ds(c * _L, _L)
                dst[r, sl] = dst[r, sl] + src[r, sl]
            return 0
        lax.fori_loop(0, _CCH, row_add, 0)

    ga0.wait()
    ga1.wait()
    add_rows(rA0, rA1)
    wa = pltpu.async_copy(rA0, out_hbm.at[pl.ds(tbase, _CCH)], swa)
    gb0.wait()
    gb1.wait()
    add_rows(rB0, rB1)
    wb = pltpu.async_copy(rB0, out_hbm.at[pl.ds(tbase + _CCH, _CCH)], swb)
    wa.wait()
    wb.wait()


@functools.partial(
    pl.kernel,
    mesh=plsc.VectorSubcoreMesh(core_axis_name="c", subcore_axis_name="s"),
    out_type=jax.ShapeDtypeStruct((T, HIDDEN), jnp.float32),
    scratch_types=[
        pltpu.VMEM((_CCH,), jnp.int32),
        pltpu.VMEM((_CCH,), jnp.int32),
        pltpu.VMEM((_CCH,), jnp.int32),
        pltpu.VMEM((_CCH,), jnp.int32),
        pltpu.VMEM((_CCH, HIDDEN), jnp.float32),
        pltpu.VMEM((_CCH, HIDDEN), jnp.float32),
        pltpu.VMEM((_CCH, HIDDEN), jnp.float32),
        pltpu.VMEM((_CCH, HIDDEN), jnp.float32),
        pltpu.SemaphoreType.DMA,
        pltpu.SemaphoreType.DMA,
        pltpu.SemaphoreType.DMA,
        pltpu.SemaphoreType.DMA,
        pltpu.SemaphoreType.DMA,
        pltpu.SemaphoreType.DMA,
    ],
)
def _sc_combine(pos_hbm, ys_hbm, out_hbm, p0a, p1a, p0b, p1b,
                rA0, rA1, rB0, rB1, sa0, sa1, sb0, sb1, swa, swb):
    _combine_body(pos_hbm, ys_hbm, out_hbm, p0a, p1a, p0b, p1b,
                  rA0, rA1, rB0, rB1, sa0, sa1, sb0, sb1, swa, swb)


@jax.jit
def kernel(hidden_states, w_router, w_gate, w_up, w_down):
    b, s, hd = hidden_states.shape
    xf = hidden_states.reshape(b * s, hd)

    pos_col, vflat_col, meta = _run_router(xf, w_router)
    pos = pos_col.reshape(NP)
    vflat = vflat_col.reshape(NP)
    bexp = meta[0, :MAXB]
    used = meta[1, :1]

    x_sorted, val_sorted = _sc_dispatch(pos, vflat, xf)
    y_sorted = _run_ffn(bexp, used, x_sorted, w_gate, w_up, w_down,
                        val_sorted.reshape(PMAX, 1))
    out = _sc_combine(pos, y_sorted)
    return out.reshape(b, s, hd)


# speedup vs baseline: 1.5144x; 1.0709x over previous
import functools

import jax
import jax.numpy as jnp
from jax import lax
from jax.experimental import pallas as pl
from jax.experimental.pallas import tpu as pltpu
from jax.experimental.pallas import tpu_sc as plsc

NUM_EXPERTS = 8
TOP_K = 2
HIDDEN = 768
INTER = 2048
T = 2048
NP = T * TOP_K
BLK = 512
MAXB = 16
PMAX = MAXB * BLK

_SC_INFO = plsc.get_sparse_core_info()
_NC, _NS, _L = _SC_INFO.num_cores, _SC_INFO.num_subcores, _SC_INFO.num_lanes
_NW = _NC * _NS


def _router_body(x_ref, wr_ref, pos_ref, vflat_ref, meta_ref):
    x = x_ref[...]
    logits = lax.dot_general(x, wr_ref[...], (((1,), (0,)), ((), ())),
                             preferred_element_type=jnp.float32)
    aff = jax.nn.sigmoid(logits)
    eiota = lax.broadcasted_iota(jnp.int32, (T, NUM_EXPERTS), 1)
    m1 = jnp.max(aff, axis=-1, keepdims=True)
    im1 = jnp.min(jnp.where(aff == m1, eiota, NUM_EXPERTS), axis=-1, keepdims=True)
    aff2 = jnp.where(eiota == im1, -jnp.inf, aff)
    m2 = jnp.max(aff2, axis=-1, keepdims=True)
    im2 = jnp.min(jnp.where(aff2 == m2, eiota, NUM_EXPERTS), axis=-1, keepdims=True)
    s = m1 + m2
    v0 = m1 / s
    v1 = m2 / s
    vflat_ref[...] = jnp.concatenate([v0, v1], axis=0)

    idx_full = jnp.concatenate([im1, im2], axis=0)
    piota = lax.broadcasted_iota(jnp.int32, (NP, NUM_EXPERTS), 1)
    onehot = (piota == idx_full).astype(jnp.float32)

    counts = jnp.sum(onehot, axis=0, keepdims=True).astype(jnp.int32)
    blocks = (counts + (BLK - 1)) >> 9
    li = lax.broadcasted_iota(jnp.int32, (NUM_EXPERTS, NUM_EXPERTS), 0)
    lj = lax.broadcasted_iota(jnp.int32, (NUM_EXPERTS, NUM_EXPERTS), 1)
    l8_incl = (li <= lj).astype(jnp.float32)
    cum_incl = lax.dot_general(blocks.astype(jnp.float32), l8_incl,
                               (((1,), (0,)), ((), ())),
                               preferred_element_type=jnp.float32).astype(jnp.int32)
    pad_off = (cum_incl - blocks) * BLK

    used = jnp.max(cum_incl)
    e_sub = lax.broadcasted_iota(jnp.int32, (1, NUM_EXPERTS), 1)
    lastexp = jnp.max(jnp.where(blocks > 0, e_sub, -1))
    counts_sub = lax.dot_general(
        onehot, jnp.ones((NP, 1), dtype=jnp.float32), (((0,), (0,)), ((), ())),
        preferred_element_type=jnp.float32).astype(jnp.int32)
    blocks_sub = (counts_sub + (BLK - 1)) >> 9
    li2 = lax.broadcasted_iota(jnp.int32, (NUM_EXPERTS, NUM_EXPERTS), 0)
    lj2 = lax.broadcasted_iota(jnp.int32, (NUM_EXPERTS, NUM_EXPERTS), 1)
    l8_incl_sub = (lj2 <= li2).astype(jnp.float32)
    cum_incl_sub = lax.dot_general(
        l8_incl_sub, blocks_sub.astype(jnp.float32), (((1,), (0,)), ((), ())),
        preferred_element_type=jnp.float32).astype(jnp.int32)
    biota = lax.broadcasted_iota(jnp.int32, (NUM_EXPERTS, 128), 1)
    be_raw = jnp.sum((cum_incl_sub <= biota).astype(jnp.int32), axis=0,
                     keepdims=True)
    meta_ref[0:1, :] = jnp.minimum(be_raw, lastexp)
    meta_ref[1:2, :] = jnp.full((1, 128), 0, jnp.int32) + used

    pad_sel = jnp.sum(onehot * pad_off.astype(jnp.float32), axis=1,
                      keepdims=True)
    tile = 512
    for tnum in range(NP // tile):
        gi = lax.broadcasted_iota(jnp.int32, (tile, NP), 0) + tnum * tile
        gj = lax.broadcasted_iota(jnp.int32, (tile, NP), 1)
        ltri = (gj < gi).astype(jnp.float32)
        rank_t = lax.dot_general(ltri, onehot, (((1,), (0,)), ((), ())),
                                 preferred_element_type=jnp.float32)
        oh_t = onehot[tnum * tile:(tnum + 1) * tile, :]
        rank_sel = jnp.sum(oh_t * rank_t, axis=1, keepdims=True)
        pos_t = pad_sel[tnum * tile:(tnum + 1) * tile, :] + rank_sel
        pos_ref[tnum * tile:(tnum + 1) * tile, :] = pos_t.astype(jnp.int32)


def _run_router(xf, w_router):
    return pl.pallas_call(
        _router_body,
        out_shape=(
            jax.ShapeDtypeStruct((NP, 1), jnp.int32),
            jax.ShapeDtypeStruct((NP, 1), jnp.float32),
            jax.ShapeDtypeStruct((2, 128), jnp.int32),
        ),
    )(xf, w_router)


_PPS = NP // _NS
_SPW = PMAX // _NW
_GCH = 64


def _dispatch_body(pos_hbm, vflat_hbm, xf_hbm, xs_hbm, val_hbm,
                   spm_tid, spm_val, posv, vv, tidv,
                   idx0, idx1, rows0, rows1,
                   gsem0, gsem1, wsem0, wsem1):
    cid = lax.axis_index("c")
    sid = lax.axis_index("s")
    wid = sid * _NC + cid

    with jax.named_scope("disp_scatter"):
        lane = lax.broadcasted_iota(jnp.int32, (_L,), 0)
        for r in range(_PPS // 128):
            base = sid * _PPS + r * 128
            pltpu.sync_copy(pos_hbm.at[pl.ds(base, 128)], posv)
            pltpu.sync_copy(vflat_hbm.at[pl.ds(base, 128)], vv)
            for i in range(128 // _L):
                tidv[pl.ds(i * _L, _L)] = (lane + (base + i * _L)) & (T - 1)
            pltpu.sync_copy(tidv, spm_tid.at[posv])
            pltpu.sync_copy(vv, spm_val.at[posv])
    with jax.named_scope("disp_barrier"):
        plsc.subcore_barrier()

    srange = wid * _SPW
    with jax.named_scope("disp_valwrite"):
        pltpu.sync_copy(spm_val.at[pl.ds(srange, _SPW)],
                        val_hbm.at[pl.ds(srange, _SPW)])

    with jax.named_scope("disp_gather"):
        idxb = (idx0, idx1)
        rows = (rows0, rows1)
        gsem = (gsem0, gsem1)
        wsem = (wsem0, wsem1)
        nch = _SPW // _GCH
        g = [None, None]
        w = [None, None]
        for c in range(nch):
            b = c & 1
            if w[b] is not None:
                w[b].wait()
            cbase = srange + c * _GCH
            pltpu.sync_copy(spm_tid.at[pl.ds(cbase, _GCH)], idxb[b])
            for i in range(_GCH // _L):
                sl = pl.ds(i * _L, _L)
                idxb[b][sl] = jnp.minimum(jnp.maximum(idxb[b][sl], 0), T - 1)
            g[b] = pltpu.async_copy(xf_hbm.at[idxb[b]], rows[b], gsem[b])
            if c >= 1:
                pb = 1 - b
                g[pb].wait()
                w[pb] = pltpu.async_copy(
                    rows[pb], xs_hbm.at[pl.ds(srange + (c - 1) * _GCH, _GCH)],
                    wsem[pb])
        lb = (nch - 1) & 1
        g[lb].wait()
        w[lb] = pltpu.async_copy(
            rows[lb], xs_hbm.at[pl.ds(srange + (nch - 1) * _GCH, _GCH)], wsem[lb])
        w[0].wait()
        w[1].wait()


@functools.partial(
    pl.kernel,
    mesh=plsc.VectorSubcoreMesh(core_axis_name="c", subcore_axis_name="s"),
    out_type=(
        jax.ShapeDtypeStruct((PMAX, HIDDEN), jnp.float32),
        jax.ShapeDtypeStruct((PMAX,), jnp.float32),
    ),
    scratch_types=[
        pltpu.VMEM_SHARED((PMAX,), jnp.int32),
        pltpu.VMEM_SHARED((PMAX,), jnp.float32),
        pltpu.VMEM((128,), jnp.int32),
        pltpu.VMEM((128,), jnp.float32),
        pltpu.VMEM((128,), jnp.int32),
        pltpu.VMEM((_GCH,), jnp.int32),
        pltpu.VMEM((_GCH,), jnp.int32),
        pltpu.VMEM((_GCH, HIDDEN), jnp.float32),
        pltpu.VMEM((_GCH, HIDDEN), jnp.float32),
        pltpu.SemaphoreType.DMA,
        pltpu.SemaphoreType.DMA,
        pltpu.SemaphoreType.DMA,
        pltpu.SemaphoreType.DMA,
    ],
)
def _sc_dispatch(pos_hbm, vflat_hbm, xf_hbm, xs_hbm, val_hbm,
                 spm_tid, spm_val, posv, vv, tidv, idx0, idx1, rows0, rows1,
                 gsem0, gsem1, wsem0, wsem1):
    _dispatch_body(pos_hbm, vflat_hbm, xf_hbm, xs_hbm, val_hbm,
                   spm_tid, spm_val, posv, vv, tidv, idx0, idx1, rows0, rows1,
                   gsem0, gsem1, wsem0, wsem1)


def _ffn_body(bexp_ref, used_ref, x_ref, wg_ref, wu_ref, wd_ref, val_ref,
              out_ref):
    b = pl.program_id(0)

    @pl.when(b < used_ref[0])
    def _compute():
        x = x_ref[...]
        g = lax.dot_general(x, wg_ref[0], (((1,), (0,)), ((), ())),
                            preferred_element_type=jnp.float32)
        u = lax.dot_general(x, wu_ref[0], (((1,), (0,)), ((), ())),
                            preferred_element_type=jnp.float32)
        h = (g * jax.nn.sigmoid(g)) * u
        h = h * val_ref[...]
        out_ref[...] = lax.dot_general(h, wd_ref[0], (((1,), (0,)), ((), ())),
                                       preferred_element_type=jnp.float32)


def _run_ffn(bexp, used, x_sorted, w_gate, w_up, w_down, val_col):
    def clamp(b, u):
        return jnp.minimum(b, u[0] - 1)

    grid_spec = pltpu.PrefetchScalarGridSpec(
        num_scalar_prefetch=2,
        grid=(MAXB,),
        in_specs=[
            pl.BlockSpec((BLK, HIDDEN), lambda b, be, u: (clamp(b, u), 0)),
            pl.BlockSpec((1, HIDDEN, INTER),
                         lambda b, be, u: (be[clamp(b, u)], 0, 0)),
            pl.BlockSpec((1, HIDDEN, INTER),
                         lambda b, be, u: (be[clamp(b, u)], 0, 0)),
            pl.BlockSpec((1, INTER, HIDDEN),
                         lambda b, be, u: (be[clamp(b, u)], 0, 0)),
            pl.BlockSpec((BLK, 1), lambda b, be, u: (clamp(b, u), 0)),
        ],
        out_specs=pl.BlockSpec((BLK, HIDDEN), lambda b, be, u: (clamp(b, u), 0)),
    )
    return pl.pallas_call(
        _ffn_body,
        grid_spec=grid_spec,
        out_shape=jax.ShapeDtypeStruct((PMAX, HIDDEN), jnp.float32),
    )(bexp, used, x_sorted, w_gate, w_up, w_down, val_col)


_TPW = T // _NW
_CCH = 32


def _combine_body(pos_hbm, ys_hbm, out_hbm, p0a, p1a, p0b, p1b,
                  rA0, rA1, rB0, rB1,
                  sa0, sa1, sb0, sb1, swa, swb):
    wid = lax.axis_index("s") * _NC + lax.axis_index("c")
    tbase = wid * _TPW
    pltpu.sync_copy(pos_hbm.at[pl.ds(tbase, _CCH)], p0a)
    pltpu.sync_copy(pos_hbm.at[pl.ds(T + tbase, _CCH)], p1a)
    ga0 = pltpu.async_copy(ys_hbm.at[p0a], rA0, sa0)
    ga1 = pltpu.async_copy(ys_hbm.at[p1a], rA1, sa1)
    pltpu.sync_copy(pos_hbm.at[pl.ds(tbase + _CCH, _CCH)], p0b)
    pltpu.sync_copy(pos_hbm.at[pl.ds(T + tbase + _CCH, _CCH)], p1b)
    gb0 = pltpu.async_copy(ys_hbm.at[p0b], rB0, sb0)
    gb1 = pltpu.async_copy(ys_hbm.at[p1b], rB1, sb1)

    def add_rows(dst, src):
        def row_add(r, _):
            for c in range(HIDDEN // _L):
                sl = pl.ds(c * _L, _L)
                dst[r, sl] = dst[r, sl] + src[r, sl]
            return 0
        lax.fori_loop(0, _CCH, row_add, 0)

    ga0.wait()
    ga1.wait()
    add_rows(rA0, rA1)
    wa = pltpu.async_copy(rA0, out_hbm.at[pl.ds(tbase, _CCH)], swa)
    gb0.wait()
    gb1.wait()
    add_rows(rB0, rB1)
    wb = pltpu.async_copy(rB0, out_hbm.at[pl.ds(tbase + _CCH, _CCH)], swb)
    wa.wait()
    wb.wait()


@functools.partial(
    pl.kernel,
    mesh=plsc.VectorSubcoreMesh(core_axis_name="c", subcore_axis_name="s"),
    out_type=jax.ShapeDtypeStruct((T, HIDDEN), jnp.float32),
    scratch_types=[
        pltpu.VMEM((_CCH,), jnp.int32),
        pltpu.VMEM((_CCH,), jnp.int32),
        pltpu.VMEM((_CCH,), jnp.int32),
        pltpu.VMEM((_CCH,), jnp.int32),
        pltpu.VMEM((_CCH, HIDDEN), jnp.float32),
        pltpu.VMEM((_CCH, HIDDEN), jnp.float32),
        pltpu.VMEM((_CCH, HIDDEN), jnp.float32),
        pltpu.VMEM((_CCH, HIDDEN), jnp.float32),
        pltpu.SemaphoreType.DMA,
        pltpu.SemaphoreType.DMA,
        pltpu.SemaphoreType.DMA,
        pltpu.SemaphoreType.DMA,
        pltpu.SemaphoreType.DMA,
        pltpu.SemaphoreType.DMA,
    ],
)
def _sc_combine(pos_hbm, ys_hbm, out_hbm, p0a, p1a, p0b, p1b,
                rA0, rA1, rB0, rB1, sa0, sa1, sb0, sb1, swa, swb):
    _combine_body(pos_hbm, ys_hbm, out_hbm, p0a, p1a, p0b, p1b,
                  rA0, rA1, rB0, rB1, sa0, sa1, sb0, sb1, swa, swb)


@jax.jit
def kernel(hidden_states, w_router, w_gate, w_up, w_down):
    b, s, hd = hidden_states.shape
    xf = hidden_states.reshape(b * s, hd)

    pos_col, vflat_col, meta = _run_router(xf, w_router)
    pos = pos_col.reshape(NP)
    vflat = vflat_col.reshape(NP)
    bexp = meta[0, :MAXB]
    used = meta[1, :1]

    x_sorted, val_sorted = _sc_dispatch(pos, vflat, xf)
    y_sorted = _run_ffn(bexp, used, x_sorted, w_gate, w_up, w_down,
                        val_sorted.reshape(PMAX, 1))
    out = _sc_combine(pos, y_sorted)
    return out.reshape(b, s, hd)


# speedup vs baseline: 2.3270x; 1.5366x over previous
import functools

import jax
import jax.numpy as jnp
from jax import lax
from jax.experimental import pallas as pl
from jax.experimental.pallas import tpu as pltpu
from jax.experimental.pallas import tpu_sc as plsc

NUM_EXPERTS = 8
TOP_K = 2
HIDDEN = 768
INTER = 2048
T = 2048
NP = T * TOP_K
BLK = 512
MAXB = 16
PMAX = MAXB * BLK

_SC_INFO = plsc.get_sparse_core_info()
_NC, _NS, _L = _SC_INFO.num_cores, _SC_INFO.num_subcores, _SC_INFO.num_lanes
_NW = _NC * _NS


def _router_body(x_ref, wr_ref, pos_ref, vflat_ref, meta_ref):
    x = x_ref[...]
    logits = lax.dot_general(x, wr_ref[...], (((1,), (0,)), ((), ())),
                             preferred_element_type=jnp.float32)
    aff = jax.nn.sigmoid(logits)
    eiota = lax.broadcasted_iota(jnp.int32, (T, NUM_EXPERTS), 1)
    m1 = jnp.max(aff, axis=-1, keepdims=True)
    im1 = jnp.min(jnp.where(aff == m1, eiota, NUM_EXPERTS), axis=-1, keepdims=True)
    aff2 = jnp.where(eiota == im1, -jnp.inf, aff)
    m2 = jnp.max(aff2, axis=-1, keepdims=True)
    im2 = jnp.min(jnp.where(aff2 == m2, eiota, NUM_EXPERTS), axis=-1, keepdims=True)
    s = m1 + m2
    v0 = m1 / s
    v1 = m2 / s
    vflat_ref[...] = jnp.concatenate([v0, v1], axis=0)

    idx_full = jnp.concatenate([im1, im2], axis=0)
    piota = lax.broadcasted_iota(jnp.int32, (NP, NUM_EXPERTS), 1)
    onehot = (piota == idx_full).astype(jnp.float32)

    counts = jnp.sum(onehot, axis=0, keepdims=True).astype(jnp.int32)
    blocks = (counts + (BLK - 1)) >> 9
    li = lax.broadcasted_iota(jnp.int32, (NUM_EXPERTS, NUM_EXPERTS), 0)
    lj = lax.broadcasted_iota(jnp.int32, (NUM_EXPERTS, NUM_EXPERTS), 1)
    l8_incl = (li <= lj).astype(jnp.float32)
    cum_incl = lax.dot_general(blocks.astype(jnp.float32), l8_incl,
                               (((1,), (0,)), ((), ())),
                               preferred_element_type=jnp.float32).astype(jnp.int32)
    pad_off = (cum_incl - blocks) * BLK

    used = jnp.max(cum_incl)
    e_sub = lax.broadcasted_iota(jnp.int32, (1, NUM_EXPERTS), 1)
    lastexp = jnp.max(jnp.where(blocks > 0, e_sub, -1))
    counts_sub = lax.dot_general(
        onehot, jnp.ones((NP, 1), dtype=jnp.float32), (((0,), (0,)), ((), ())),
        preferred_element_type=jnp.float32).astype(jnp.int32)
    blocks_sub = (counts_sub + (BLK - 1)) >> 9
    li2 = lax.broadcasted_iota(jnp.int32, (NUM_EXPERTS, NUM_EXPERTS), 0)
    lj2 = lax.broadcasted_iota(jnp.int32, (NUM_EXPERTS, NUM_EXPERTS), 1)
    l8_incl_sub = (lj2 <= li2).astype(jnp.float32)
    cum_incl_sub = lax.dot_general(
        l8_incl_sub, blocks_sub.astype(jnp.float32), (((1,), (0,)), ((), ())),
        preferred_element_type=jnp.float32).astype(jnp.int32)
    biota = lax.broadcasted_iota(jnp.int32, (NUM_EXPERTS, 128), 1)
    be_raw = jnp.sum((cum_incl_sub <= biota).astype(jnp.int32), axis=0,
                     keepdims=True)
    meta_ref[0:1, :] = jnp.minimum(be_raw, lastexp)
    meta_ref[1:2, :] = jnp.full((1, 128), 0, jnp.int32) + used

    pad_sel = jnp.sum(onehot * pad_off.astype(jnp.float32), axis=1,
                      keepdims=True)
    tile = 512
    for tnum in range(NP // tile):
        gi = lax.broadcasted_iota(jnp.int32, (tile, NP), 0) + tnum * tile
        gj = lax.broadcasted_iota(jnp.int32, (tile, NP), 1)
        ltri = (gj < gi).astype(jnp.float32)
        rank_t = lax.dot_general(ltri, onehot, (((1,), (0,)), ((), ())),
                                 preferred_element_type=jnp.float32)
        oh_t = onehot[tnum * tile:(tnum + 1) * tile, :]
        rank_sel = jnp.sum(oh_t * rank_t, axis=1, keepdims=True)
        pos_t = pad_sel[tnum * tile:(tnum + 1) * tile, :] + rank_sel
        pos_ref[tnum * tile:(tnum + 1) * tile, :] = pos_t.astype(jnp.int32)


def _run_router(xf, w_router):
    return pl.pallas_call(
        _router_body,
        out_shape=(
            jax.ShapeDtypeStruct((NP, 1), jnp.int32),
            jax.ShapeDtypeStruct((NP, 1), jnp.float32),
            jax.ShapeDtypeStruct((2, 128), jnp.int32),
        ),
    )(xf, w_router)


def _ffn_body(bexp_ref, used_ref, posr_ref, vr_ref, xf_ref,
              wg_ref, wu_ref, wd_ref, out_ref):
    b = pl.program_id(0)

    @pl.when(b < used_ref[0])
    def _compute():
        sl_iota = lax.broadcasted_iota(jnp.int32, (BLK, T), 0) + b * BLK
        p0 = (sl_iota == posr_ref[0:1, :]).astype(jnp.float32)
        p1 = (sl_iota == posr_ref[1:2, :]).astype(jnp.float32)
        perm = p0 + p1
        x = lax.dot_general(perm, xf_ref[...], (((1,), (0,)), ((), ())),
                            preferred_element_type=jnp.float32)
        val = (lax.dot_general(p0, vr_ref[0:1, :], (((1,), (1,)), ((), ())),
                               preferred_element_type=jnp.float32)
               + lax.dot_general(p1, vr_ref[1:2, :], (((1,), (1,)), ((), ())),
                                 preferred_element_type=jnp.float32))
        g = lax.dot_general(x, wg_ref[0], (((1,), (0,)), ((), ())),
                            preferred_element_type=jnp.float32)
        u = lax.dot_general(x, wu_ref[0], (((1,), (0,)), ((), ())),
                            preferred_element_type=jnp.float32)
        h = (g * jax.nn.sigmoid(g)) * u * val
        out_ref[...] = lax.dot_general(h, wd_ref[0], (((1,), (0,)), ((), ())),
                                       preferred_element_type=jnp.float32)


def _run_ffn(bexp, used, posr, vr, xf, w_gate, w_up, w_down):
    def clamp(b, u):
        return jnp.minimum(b, u[0] - 1)

    grid_spec = pltpu.PrefetchScalarGridSpec(
        num_scalar_prefetch=2,
        grid=(MAXB,),
        in_specs=[
            pl.BlockSpec((8, T), lambda b, be, u: (0, 0)),
            pl.BlockSpec((8, T), lambda b, be, u: (0, 0)),
            pl.BlockSpec((T, HIDDEN), lambda b, be, u: (0, 0)),
            pl.BlockSpec((1, HIDDEN, INTER),
                         lambda b, be, u: (be[clamp(b, u)], 0, 0)),
            pl.BlockSpec((1, HIDDEN, INTER),
                         lambda b, be, u: (be[clamp(b, u)], 0, 0)),
            pl.BlockSpec((1, INTER, HIDDEN),
                         lambda b, be, u: (be[clamp(b, u)], 0, 0)),
        ],
        out_specs=pl.BlockSpec((BLK, HIDDEN), lambda b, be, u: (clamp(b, u), 0)),
    )
    return pl.pallas_call(
        _ffn_body,
        grid_spec=grid_spec,
        out_shape=jax.ShapeDtypeStruct((PMAX, HIDDEN), jnp.float32),
    )(bexp, used, posr, vr, xf, w_gate, w_up, w_down)


_TPW = T // _NW
_CCH = 32


def _combine_body(pos_hbm, ys_hbm, out_hbm, p0a, p1a, p0b, p1b,
                  rA0, rA1, rB0, rB1,
                  sa0, sa1, sb0, sb1, swa, swb):
    wid = lax.axis_index("s") * _NC + lax.axis_index("c")
    tbase = wid * _TPW
    pltpu.sync_copy(pos_hbm.at[pl.ds(tbase, _CCH)], p0a)
    pltpu.sync_copy(pos_hbm.at[pl.ds(T + tbase, _CCH)], p1a)
    ga0 = pltpu.async_copy(ys_hbm.at[p0a], rA0, sa0)
    ga1 = pltpu.async_copy(ys_hbm.at[p1a], rA1, sa1)
    pltpu.sync_copy(pos_hbm.at[pl.ds(tbase + _CCH, _CCH)], p0b)
    pltpu.sync_copy(pos_hbm.at[pl.ds(T + tbase + _CCH, _CCH)], p1b)
    gb0 = pltpu.async_copy(ys_hbm.at[p0b], rB0, sb0)
    gb1 = pltpu.async_copy(ys_hbm.at[p1b], rB1, sb1)

    def add_rows(dst, src):
        def row_add(r, _):
            for c in range(HIDDEN // _L):
                sl = pl.ds(c * _L, _L)
                dst[r, sl] = dst[r, sl] + src[r, sl]
            return 0
        lax.fori_loop(0, _CCH, row_add, 0)

    ga0.wait()
    ga1.wait()
    add_rows(rA0, rA1)
    wa = pltpu.async_copy(rA0, out_hbm.at[pl.ds(tbase, _CCH)], swa)
    gb0.wait()
    gb1.wait()
    add_rows(rB0, rB1)
    wb = pltpu.async_copy(rB0, out_hbm.at[pl.ds(tbase + _CCH, _CCH)], swb)
    wa.wait()
    wb.wait()


@functools.partial(
    pl.kernel,
    mesh=plsc.VectorSubcoreMesh(core_axis_name="c", subcore_axis_name="s"),
    out_type=jax.ShapeDtypeStruct((T, HIDDEN), jnp.float32),
    scratch_types=[
        pltpu.VMEM((_CCH,), jnp.int32),
        pltpu.VMEM((_CCH,), jnp.int32),
        pltpu.VMEM((_CCH,), jnp.int32),
        pltpu.VMEM((_CCH,), jnp.int32),
        pltpu.VMEM((_CCH, HIDDEN), jnp.float32),
        pltpu.VMEM((_CCH, HIDDEN), jnp.float32),
        pltpu.VMEM((_CCH, HIDDEN), jnp.float32),
        pltpu.VMEM((_CCH, HIDDEN), jnp.float32),
        pltpu.SemaphoreType.DMA,
        pltpu.SemaphoreType.DMA,
        pltpu.SemaphoreType.DMA,
        pltpu.SemaphoreType.DMA,
        pltpu.SemaphoreType.DMA,
        pltpu.SemaphoreType.DMA,
    ],
)
def _sc_combine(pos_hbm, ys_hbm, out_hbm, p0a, p1a, p0b, p1b,
                rA0, rA1, rB0, rB1, sa0, sa1, sb0, sb1, swa, swb):
    _combine_body(pos_hbm, ys_hbm, out_hbm, p0a, p1a, p0b, p1b,
                  rA0, rA1, rB0, rB1, sa0, sa1, sb0, sb1, swa, swb)


@jax.jit
def kernel(hidden_states, w_router, w_gate, w_up, w_down):
    b, s, hd = hidden_states.shape
    xf = hidden_states.reshape(b * s, hd)

    pos_col, vflat_col, meta = _run_router(xf, w_router)
    posr = pos_col.reshape(2, T)
    vr = vflat_col.reshape(2, T)
    posr8 = jnp.concatenate([posr, jnp.zeros((6, T), jnp.int32)], axis=0)
    vr8 = jnp.concatenate([vr, jnp.zeros((6, T), jnp.float32)], axis=0)
    bexp = meta[0, :MAXB]
    used = meta[1, :1]

    y_sorted = _run_ffn(bexp, used, posr8, vr8, xf, w_gate, w_up, w_down)
    out = _sc_combine(posr.reshape(NP), y_sorted)
    return out.reshape(b, s, hd)
